# Initial kernel scaffold; baseline (speedup 1.0000x reference)
import jax
import jax.numpy as jnp
from jax.experimental import pallas as pl


def kernel(x, w_router, w1, w2):
    raise NotImplementedError("write your pallas kernel here")


# speedup vs baseline: 1.5204x; 1.5204x over previous
import jax
import jax.numpy as jnp
from jax import lax
from jax.experimental import pallas as pl
from jax.experimental.pallas import tpu as pltpu

SL, BS, HS = 2048, 1, 1024
E, TOPK, FFN = 64, 2, 1024
T = SL * BS
CAP = (TOPK * T) // E
CHUNK = 256
NCHUNK = T // CHUNK


def _dot_t(a, b, precision=lax.Precision.HIGHEST):
    return lax.dot_general(a, b, (((0,), (0,)), ((), ())),
                           preferred_element_type=jnp.float32,
                           precision=precision)


def _routing_step(xf_ref, wr_ref, gidx_ref, gw_ref, b_ref, s_ref):
    logits = jnp.dot(xf_ref[...], wr_ref[...], preferred_element_type=jnp.float32,
                     precision=lax.Precision.DEFAULT)
    m = jnp.max(logits, axis=1, keepdims=True)
    p = jnp.exp(logits - m)
    scores = p / jnp.sum(p, axis=1, keepdims=True)
    iota_e = lax.broadcasted_iota(jnp.int32, (T, E), 1)
    m1 = jnp.max(scores, axis=1, keepdims=True)
    e1 = jnp.min(jnp.where(scores == m1, iota_e, E), axis=1, keepdims=True)
    a1 = (iota_e == e1).astype(jnp.float32)
    scores2 = jnp.where(iota_e == e1, -1.0, scores)
    m2 = jnp.max(scores2, axis=1, keepdims=True)
    e2 = jnp.min(jnp.where(scores2 == m2, iota_e, E), axis=1, keepdims=True)
    a2 = (iota_e == e2).astype(jnp.float32)
    b_ref[...] = a1 + a2
    li = lax.broadcasted_iota(jnp.int32, (CHUNK, CHUNK), 0)
    lj = lax.broadcasted_iota(jnp.int32, (CHUNK, CHUNK), 1)
    lstrict = (li > lj).astype(jnp.float32)

    def body(c, run):
        bc = b_ref[pl.ds(c * CHUNK, CHUNK), :]
        s_ref[pl.ds(c * CHUNK, CHUNK), :] = (
            jnp.dot(lstrict, bc, preferred_element_type=jnp.float32) + run)
        return run + jnp.sum(bc, axis=0, keepdims=True)

    lax.fori_loop(0, NCHUNK, body, jnp.zeros((1, E), jnp.float32))
    s = s_ref[...]
    r1 = jnp.sum(s * a1, axis=1, keepdims=True)
    r2 = jnp.sum(s * a2, axis=1, keepdims=True)
    iota_cap = lax.broadcasted_iota(jnp.int32, (T, CAP), 1)
    r1i = r1.astype(jnp.int32)
    r2i = r2.astype(jnp.int32)
    rr1 = ((iota_cap == r1i) & (r1i < CAP)).astype(jnp.float32)
    rr2 = ((iota_cap == r2i) & (r2i < CAP)).astype(jnp.float32)
    tcol = lax.broadcasted_iota(jnp.int32, (T, 1), 0).astype(jnp.float32)
    occ = _dot_t(a1, rr1, lax.Precision.DEFAULT) + _dot_t(a2, rr2, lax.Precision.DEFAULT)
    gi = _dot_t(a1 * tcol, rr1) + _dot_t(a2 * tcol, rr2)
    gv = _dot_t(a1 * m1, rr1) + _dot_t(a2 * m2, rr2)
    filled = occ > 0.5
    gidx_ref[...] = jnp.where(filled, gi.astype(jnp.int32), -1)
    gw_ref[...] = jnp.where(filled, gv, 0.0)


def _expert_step(ei, xf_ref, w1_ref, w2_ref, y_ref, gidx_ref, gw_ref):
    g = gidx_ref[pl.ds(ei, 1), :]
    gwrow = gw_ref[pl.ds(ei, 1), :]
    iota_t = lax.broadcasted_iota(jnp.int32, (T, CAP), 0)
    pt = ((iota_t == g) & (g >= 0)).astype(jnp.float32)
    xe = _dot_t(pt, xf_ref[...], lax.Precision.DEFAULT)
    h = jnp.maximum(jnp.dot(xe, w1_ref[0], preferred_element_type=jnp.float32,
                            precision=lax.Precision.DEFAULT), 0.0)
    yo = jnp.dot(h, w2_ref[0], preferred_element_type=jnp.float32,
                 precision=lax.Precision.DEFAULT)
    ptw = pt * gwrow
    contrib = jnp.dot(ptw, yo, preferred_element_type=jnp.float32,
                      precision=lax.Precision.DEFAULT)

    @pl.when(ei == 0)
    def _():
        y_ref[...] = contrib

    @pl.when(ei != 0)
    def _():
        y_ref[...] = y_ref[...] + contrib


def _moe_kernel(xf_ref, wr_ref, w1_ref, w2_ref, y_ref,
                gidx_ref, gw_ref, b_ref, s_ref):
    i = pl.program_id(0)

    @pl.when(i == 0)
    def _():
        _routing_step(xf_ref, wr_ref, gidx_ref, gw_ref, b_ref, s_ref)

    @pl.when(i > 0)
    def _():
        _expert_step(i - 1, xf_ref, w1_ref, w2_ref, y_ref, gidx_ref, gw_ref)


def _moe(xf, w_router, w1, w2, interpret=False):
    return pl.pallas_call(
        _moe_kernel,
        grid=(E + 1,),
        out_shape=jax.ShapeDtypeStruct((T, HS), jnp.float32),
        in_specs=[
            pl.BlockSpec((T, HS), lambda i: (0, 0)),
            pl.BlockSpec((HS, E), lambda i: (0, 0)),
            pl.BlockSpec((1, HS, FFN), lambda i: (jnp.maximum(i - 1, 0), 0, 0)),
            pl.BlockSpec((1, FFN, HS), lambda i: (jnp.maximum(i - 1, 0), 0, 0)),
        ],
        out_specs=pl.BlockSpec((T, HS), lambda i: (0, 0)),
        scratch_shapes=[
            pltpu.VMEM((E, CAP), jnp.int32),
            pltpu.VMEM((E, CAP), jnp.float32),
            pltpu.VMEM((T, E), jnp.float32),
            pltpu.VMEM((T, E), jnp.float32),
        ],
        compiler_params=pltpu.CompilerParams(
            dimension_semantics=("arbitrary",)),
        interpret=interpret,
    )(xf, w_router, w1, w2)


def kernel(x, w_router, w1, w2):
    xf = x.reshape(T, HS)
    y = _moe(xf, w_router, w1, w2)
    return y.reshape(SL, BS, HS)
